# Initial kernel scaffold; baseline (speedup 1.0000x reference)
import jax
import jax.numpy as jnp
from jax.experimental import pallas as pl


def kernel(h_hyper, weight, loop_weight, evolve_loop_weight, edge_index, edge_type):
    raise NotImplementedError("write your pallas kernel here")


# speedup vs baseline: 23.4439x; 23.4439x over previous
import functools

import jax
import jax.numpy as jnp
import numpy as np
from jax import lax
from jax.experimental import pallas as pl
from jax.experimental.pallas import tpu as pltpu
from jax.experimental.pallas import tpu_sc as plsc

_C = 0.01
_SQRT_C = 0.1
_N = 10000
_E = 320000
_D = 128
_R = 200
_EPS = 1e-15

_ACC_W = 136
_CH = 128
_NCH = _E // _CH
_NHALF = _N // 2
_ACC_ROWS = _NHALF + 8


def _stage1_body(h_ref, lw_ref, elw_ref, htan_ref, lin_ref, lall_ref):
    h = h_ref[...]
    n = jnp.sqrt(jnp.sum(h * h, axis=-1, keepdims=True) + _EPS)
    scn = jnp.clip(_SQRT_C * n, 0.0, 1.0 - 1e-5)
    at = 0.5 * jnp.log((1.0 + scn) / (1.0 - scn))
    htan = at * h / (_SQRT_C * n)
    htan_ref[...] = htan
    lin_ref[...] = jnp.dot(htan, lw_ref[...], preferred_element_type=jnp.float32)
    lall_ref[...] = jnp.dot(htan, elw_ref[...], preferred_element_type=jnp.float32)


def _stage1(h_perm, lw_pp, elw_pp):
    return pl.pallas_call(
        _stage1_body,
        out_shape=[
            jax.ShapeDtypeStruct((_N, _D), jnp.float32),
            jax.ShapeDtypeStruct((_N, _D), jnp.float32),
            jax.ShapeDtypeStruct((_N, _D), jnp.float32),
        ],
    )(h_perm, lw_pp, elw_pp)


def _rsqrt_nr(x):
    i = lax.bitcast_convert_type(x, jnp.int32)
    i = jnp.int32(0x5F3759DF) - lax.shift_right_arithmetic(i, 1)
    y = lax.bitcast_convert_type(i, jnp.float32)
    for _ in range(3):
        y = y * (1.5 - 0.5 * x * y * y)
    return y


_sc_mesh = plsc.VectorSubcoreMesh(core_axis_name="c", subcore_axis_name="s")


@functools.partial(
    pl.kernel,
    mesh=_sc_mesh,
    compiler_params=pltpu.CompilerParams(
        needs_layout_passes=False, use_tc_tiling_on_sc=False),
    out_type=jax.ShapeDtypeStruct((2, _NHALF, _ACC_W), jnp.float32),
    scratch_types=[
        pltpu.VMEM((_R, 256), jnp.float32),
        pltpu.VMEM((_CH,), jnp.int32),
        pltpu.VMEM((_CH,), jnp.int32),
        pltpu.VMEM((_CH,), jnp.int32),
        pltpu.VMEM((_CH,), jnp.int32),
        pltpu.VMEM((_CH, _D), jnp.float32),
        pltpu.VMEM((_CH, _ACC_W), jnp.float32),
        pltpu.VMEM((_D, 16), jnp.float32),
        pltpu.VMEM_SHARED((_ACC_ROWS, _ACC_W), jnp.float32),
        pltpu.SemaphoreType.DMA,
    ],
)
def _edge_kernel(h_hbm, wt_hbm, src_hbm, dst_hbm, typ_hbm, out_hbm,
                 wt_v, src_v, dst_v, dloc_v, typ_v, hrows, obuf, mbuf,
                 acc, sem):
    c = lax.axis_index("c")
    s = lax.axis_index("s")
    lane = lax.iota(jnp.int32, 16)
    zero = jnp.zeros((16,), jnp.float32)

    pltpu.sync_copy(wt_hbm, wt_v)

    def _zrow(i, _):
        for j in range(_ACC_W // 16):
            obuf[i, pl.ds(j * 16, 16)] = zero
        obuf[i, pl.ds(_ACC_W - 16, 16)] = zero
        return 0

    lax.fori_loop(0, _CH, _zrow, 0)

    def _zblock(k, _):
        bid = s + k * 16

        @pl.when(bid < 39)
        def _():
            pltpu.sync_copy(obuf, acc.at[pl.ds(bid * _CH, _CH)])

        return 0

    lax.fori_loop(0, 3, _zblock, 0)

    @pl.when(s == 15)
    def _():
        pltpu.sync_copy(obuf.at[pl.ds(0, 16)], acc.at[pl.ds(39 * _CH, 16)])

    cvec = 128 + (lane & 7)
    cmask = lane < 8
    cpat = jnp.where(lane == 1, 1.0, 0.0).astype(jnp.float32)

    def _crow(i, _):
        plsc.store_scatter(obuf, [jnp.full((16,), i, jnp.int32), cvec], cpat,
                           mask=cmask)
        return 0

    lax.fori_loop(0, _CH, _crow, 0)
    plsc.subcore_barrier()

    nbase = c * _NHALF

    def _edge_group(g, _):
        eids = g * 16 + lane
        tvec = typ_v[pl.ds(g * 16, 16)]

        def _dim1(k, accv):
            c0 = jnp.full((16,), k, jnp.int32)
            he = plsc.load_gather(hrows, [eids, c0])
            ho = plsc.load_gather(hrows, [eids, c0 + 64])
            w00 = plsc.load_gather(wt_v, [tvec, c0])
            w01 = plsc.load_gather(wt_v, [tvec, c0 + 64])
            w10 = plsc.load_gather(wt_v, [tvec, c0 + 128])
            w11 = plsc.load_gather(wt_v, [tvec, c0 + 192])
            me = he * w00 + ho * w10
            mo = he * w01 + ho * w11
            mbuf[k] = me
            mbuf[64 + k] = mo
            return accv + me * me + mo * mo

        accv = lax.fori_loop(0, 64, _dim1, zero)
        x = accv + _EPS
        rn = _rsqrt_nr(x)
        n = x * rn
        ex = jnp.exp(0.2 * n)
        th = (ex - 1.0) / (ex + 1.0)
        scale = 10.0 * th * rn
        r2p = accv * scale * scale
        den = jnp.maximum(1.0 - _C * r2p, 1e-6)
        tco = (1.0 + _C * r2p) / (_SQRT_C * den)
        ss = 2.0 * scale / den

        def _dim2(k, _):
            c0 = jnp.full((16,), k, jnp.int32)
            plsc.store_scatter(obuf, [eids, c0], mbuf[k] * ss)
            plsc.store_scatter(obuf, [eids, c0 + 64], mbuf[64 + k] * ss)
            return 0

        lax.fori_loop(0, 64, _dim2, 0)
        plsc.store_scatter(obuf, [eids, jnp.full((16,), 128, jnp.int32)], tco)

        dvec = dst_v[pl.ds(g * 16, 16)] - nbase
        ok = (dvec >= 0) & (dvec < _NHALF)
        dloc_v[pl.ds(g * 16, 16)] = jnp.where(ok, dvec, _NHALF)
        return 0

    def _chunk_body(k, _):
        cid = s + k * 16

        @pl.when(cid < _NCH)
        def _():
            base = cid * _CH
            pltpu.sync_copy(src_hbm.at[pl.ds(base, _CH)], src_v)
            pltpu.sync_copy(dst_hbm.at[pl.ds(base, _CH)], dst_v)
            pltpu.sync_copy(typ_hbm.at[pl.ds(base, _CH)], typ_v)
            pltpu.async_copy(h_hbm.at[src_v], hrows, sem).wait()
            lax.fori_loop(0, _CH // 16, _edge_group, 0)
            pltpu.sync_copy(obuf, acc.at[dloc_v], add=True)

        return 0

    lax.fori_loop(0, (_NCH + 15) // 16, _chunk_body, 0)
    plsc.subcore_barrier()

    def _wblock(k, _):
        bid = s + k * 16

        @pl.when(bid < 39)
        def _():
            pltpu.sync_copy(acc.at[pl.ds(bid * _CH, _CH)],
                            out_hbm.at[c, pl.ds(bid * _CH, _CH)])

        return 0

    lax.fori_loop(0, 3, _wblock, 0)

    @pl.when(s == 15)
    def _():
        pltpu.sync_copy(acc.at[pl.ds(39 * _CH, 8)],
                        out_hbm.at[c, pl.ds(39 * _CH, 8)])


def _stage3_body(s_ref, td_ref, lin_ref, lall_ref, out_ref):
    mu_s = s_ref[...]
    td = td_ref[...]
    tco = td[:, 0:1]
    deg = td[:, 1:2]
    w = 1.0 / (deg + 1e-6)
    mu_sp = mu_s * w
    mu_t = tco * w
    inner = -(mu_t * mu_t) + jnp.sum(mu_sp * mu_sp, axis=-1, keepdims=True)
    denom = _SQRT_C * jnp.sqrt(jnp.maximum(-inner, 1e-10))
    hl_t = mu_t / denom
    hl_s = mu_sp / denom
    p = hl_s / (1.0 + _SQRT_C * hl_t + 1e-15)
    n = jnp.sqrt(jnp.sum(p * p, axis=-1, keepdims=True) + _EPS)
    scn = jnp.clip(_SQRT_C * n, 0.0, 1.0 - 1e-5)
    at = 0.5 * jnp.log((1.0 + scn) / (1.0 - scn))
    h_new = jnp.clip(at * p / (_SQRT_C * n), -10.0, 10.0)
    loop_msg = jnp.where(deg > 0.0, lin_ref[...], lall_ref[...])
    h_new = jnp.clip(h_new + loop_msg, -10.0, 10.0)
    n2 = jnp.sqrt(jnp.sum(h_new * h_new, axis=-1, keepdims=True) + _EPS)
    out_ref[...] = jnp.tanh(_SQRT_C * n2) * h_new / (_SQRT_C * n2)


def _stage3(s_part, td_part, lin, lall):
    return pl.pallas_call(
        _stage3_body,
        out_shape=jax.ShapeDtypeStruct((_N, _D), jnp.float32),
    )(s_part, td_part, lin, lall)


def kernel(h_hyper, weight, loop_weight, evolve_loop_weight, edge_index, edge_type):
    perm = np.concatenate([np.arange(0, _D, 2), np.arange(1, _D, 2)])
    h_perm = h_hyper[:, perm]
    lw_pp = loop_weight[perm][:, perm]
    elw_pp = evolve_loop_weight[perm][:, perm]
    wt = weight.reshape(_R, 64, 4).transpose(0, 2, 1).reshape(_R, 256)

    htan_p, lin_p, lall_p = _stage1(h_perm, lw_pp, elw_pp)
    acc = _edge_kernel(htan_p, wt, edge_index[0], edge_index[1], edge_type)
    full = acc.reshape(_N, _ACC_W)
    out_p = _stage3(full[:, :_D], full[:, _D:], lin_p, lall_p)
    return out_p.reshape(_N, 2, 64).transpose(0, 2, 1).reshape(_N, _D)
